# baseline (speedup 1.0000x reference)
import functools

import jax
import jax.numpy as jnp
from jax import lax
from jax.experimental import pallas as pl
from jax.experimental.pallas import tpu as pltpu
from jax.experimental.pallas import tpu_sc as plsc

_B, _L, _E, _OV = 16384, 200, 64, 256
_IV = 1_000_000

_NW = 32
_BPW = _B // _NW
_CHUNK = 2
_T = _BPW // _CHUNK
_SUBLEN = 100
_NSUB = (_CHUNK * _L) // _SUBLEN
_GROUPS = _E // 16
_UNROLL = 8


_CB = 128


def _relayout_tc(tabT):
    iota_r = jax.lax.broadcasted_iota(jnp.int32, (_CB, _CB // 2), 0)
    iota_c = jax.lax.broadcasted_iota(jnp.int32, (_CB, _CB // 2), 1)
    p_even = (iota_r == 2 * iota_c).astype(jnp.float32)
    p_odd = (iota_r == 2 * iota_c + 1).astype(jnp.float32)

    def body(pe_ref, po_ref, in_ref, out_ref):
        x = in_ref[...]
        a = jax.lax.dot_general(pe_ref[...], x, (((0,), (1,)), ((), ())),
                                preferred_element_type=jnp.float32)
        b = jax.lax.dot_general(po_ref[...], x, (((0,), (1,)), ((), ())),
                                preferred_element_type=jnp.float32)
        out_ref[...] = jnp.concatenate([a, b], axis=1)

    return pl.pallas_call(
        body,
        grid=(_IV // _CB,),
        in_specs=[
            pl.BlockSpec((_CB, _CB // 2), lambda j: (0, 0)),
            pl.BlockSpec((_CB, _CB // 2), lambda j: (0, 0)),
            pl.BlockSpec((_E, _CB), lambda j: (0, j)),
        ],
        out_specs=pl.BlockSpec((_CB // 2, 128), lambda j: (j, 0)),
        out_shape=jax.ShapeDtypeStruct((_IV // 2, 128), jnp.float32),
    )(p_even, p_odd, tabT)


def _embed_sum_sc(idx3, emb_table):
    mesh = plsc.VectorSubcoreMesh(core_axis_name="c", subcore_axis_name="s")

    @functools.partial(
        pl.kernel,
        out_type=jax.ShapeDtypeStruct((_B, _E), jnp.float32),
        mesh=mesh,
        scratch_types=[
            pltpu.VMEM((2, _NSUB, _SUBLEN), jnp.int32),
            pltpu.VMEM((2, _CHUNK * _L, _E), jnp.float32),
            pltpu.VMEM((_BPW, _E), jnp.float32),
            pltpu.SemaphoreType.DMA,
            pltpu.SemaphoreType.DMA,
        ],
        compiler_params=pltpu.CompilerParams(use_tc_tiling_on_sc=False),
    )
    def k(idx_hbm, table_hbm, out_hbm, idx_v, rows_v, out_v, sem0, sem1):
        wid = lax.axis_index("s") * 2 + lax.axis_index("c")
        row0 = wid * _T

        def fire(t, buf, sem):
            pltpu.sync_copy(idx_hbm.at[row0 + t], idx_v.at[buf])
            for j in range(_NSUB):
                pltpu.async_copy(
                    table_hbm.at[idx_v.at[buf, j]],
                    rows_v.at[buf, pl.ds(j * _SUBLEN, _SUBLEN)],
                    sem,
                )

        def drain(buf, sem):
            for j in range(_NSUB):
                pltpu.make_async_copy(
                    table_hbm.at[idx_v.at[buf, j]],
                    rows_v.at[buf, pl.ds(j * _SUBLEN, _SUBLEN)],
                    sem,
                ).wait()

        def accum(t, buf):
            for c in range(_CHUNK):
                base = c * _L

                def body(jj, accs):
                    j = base + jj * _UNROLL
                    new = []
                    for g in range(_GROUPS):
                        a = accs[g]
                        for u in range(_UNROLL):
                            a = a + rows_v[buf, j + u, pl.ds(g * 16, 16)]
                        new.append(a)
                    return tuple(new)

                zero = jnp.zeros((16,), jnp.float32)
                accs = lax.fori_loop(0, _L // _UNROLL, body, (zero,) * _GROUPS)
                for g in range(_GROUPS):
                    out_v[t * _CHUNK + c, pl.ds(g * 16, 16)] = accs[g]

        fire(0, 0, sem0)

        def pair(i, carry):
            t0 = 2 * i
            fire(t0 + 1, 1, sem1)
            drain(0, sem0)
            accum(t0, 0)

            @pl.when(t0 + 2 < _T)
            def _():
                fire(t0 + 2, 0, sem0)

            drain(1, sem1)
            accum(t0 + 1, 1)
            return carry

        lax.fori_loop(0, _T // 2, pair, 0)
        pltpu.sync_copy(out_v, out_hbm.at[pl.ds(wid * _BPW, _BPW)])

    return k(idx3, emb_table)


def _heads_tc(hid, w_outT, b_out2, w_val2, b_val2, u):
    blk = 512
    grid = _B // blk

    def body(hid_ref, w_ref, b_ref, wv_ref, bv_ref, u_ref,
             comms_ref, probs_ref, val_ref):
        h = hid_ref[...]
        logits = jnp.dot(h, w_ref[...], preferred_element_type=jnp.float32)
        logits = logits + b_ref[...]
        p = jax.nn.sigmoid(logits)
        probs_ref[...] = p
        comms_ref[...] = (u_ref[...] < p).astype(jnp.float32)
        v = jnp.sum(h * wv_ref[...], axis=1, keepdims=True) + bv_ref[0, 0]
        val_ref[...] = v

    return pl.pallas_call(
        body,
        grid=(grid,),
        in_specs=[
            pl.BlockSpec((blk, _E), lambda i: (i, 0)),
            pl.BlockSpec((_E, _OV), lambda i: (0, 0)),
            pl.BlockSpec((1, _OV), lambda i: (0, 0)),
            pl.BlockSpec((1, _E), lambda i: (0, 0)),
            pl.BlockSpec(memory_space=pltpu.SMEM),
            pl.BlockSpec((blk, _OV), lambda i: (i, 0)),
        ],
        out_specs=[
            pl.BlockSpec((blk, _OV), lambda i: (i, 0)),
            pl.BlockSpec((blk, _OV), lambda i: (i, 0)),
            pl.BlockSpec((blk, 1), lambda i: (i, 0)),
        ],
        out_shape=[
            jax.ShapeDtypeStruct((_B, _OV), jnp.float32),
            jax.ShapeDtypeStruct((_B, _OV), jnp.float32),
            jax.ShapeDtypeStruct((_B, 1), jnp.float32),
        ],
    )(hid, w_outT, b_out2, w_val2, b_val2, u)


def kernel(goldstandard, emb_table, W_out, b_out, W_val, b_val):
    idx3 = goldstandard.reshape(_B // _CHUNK, _NSUB, _SUBLEN)
    table_lin = _relayout_tc(emb_table.T).reshape(_IV, _E)
    hid = _embed_sum_sc(idx3, table_lin)
    u = jax.random.uniform(jax.random.key(1), (_B, _OV), jnp.float32)
    comms, probs, value = _heads_tc(
        hid,
        W_out.T,
        b_out.reshape(1, _OV),
        W_val,
        b_val.reshape(1, 1),
        u,
    )
    return comms, probs, value


# speedup vs baseline: 3.3127x; 3.3127x over previous
import functools

import jax
import jax.numpy as jnp
from jax import lax
from jax.experimental import pallas as pl
from jax.experimental.pallas import tpu as pltpu
from jax.experimental.pallas import tpu_sc as plsc

_B, _L, _E, _OV = 16384, 200, 64, 256
_IV = 1_000_000

_NW = 32
_BPW = _B // _NW
_CHUNK = 2
_T = _BPW // _CHUNK
_SUBLEN = 100
_NSUB = (_CHUNK * _L) // _SUBLEN
_GROUPS = _E // 16
_UNROLL = 8


_CB = 1024
_NGR = 977
_TAIL = 512
_REM = _IV - (_NGR - 1) * _CB - _TAIL


def _relayout_tc(tabT):
    def body(in_hbm, out_hbm, x0, x1, s0, s1, si0, si1, so0, so1):
        j = pl.program_id(0)

        def in_desc(jj, xbuf, sem, n):
            return pltpu.make_async_copy(
                in_hbm.at[:, pl.ds(jj * _CB, n)], xbuf.at[:, pl.ds(0, n)], sem)

        def out_desc(jj, sbuf, sem, n):
            return pltpu.make_async_copy(
                sbuf.at[pl.ds(0, n), :], out_hbm.at[pl.ds(jj * _CB, n), :], sem)

        def step(x, si, s, so, xn, sin_, sprev, soprev):
            @pl.when(j == 0)
            def _():
                in_desc(0, x, si, _CB).start()

            @pl.when(j < _NGR - 1)
            def _():
                in_desc(j, x, si, _CB).wait()

            @pl.when(j == _NGR - 1)
            def _():
                in_desc(j, x, si, _TAIL).wait()

            @pl.when(j + 1 < _NGR - 1)
            def _():
                in_desc(j + 1, xn, sin_, _CB).start()

            @pl.when(j + 1 == _NGR - 1)
            def _():
                in_desc(j + 1, xn, sin_, _TAIL).start()

            @pl.when(j >= 2)
            def _():
                out_desc(j - 2, s, so, _CB).wait()

            @pl.when(j < _NGR - 1)
            def _():
                s[:, 0:_E] = jnp.transpose(x[...], (1, 0))
                out_desc(j, s, so, _CB).start()

            @pl.when(j == _NGR - 1)
            def _():
                s[0:_TAIL, 0:_E] = jnp.transpose(x[:, 0:_TAIL], (1, 0))
                out_desc(j, s, so, _TAIL).start()
                out_desc(j, s, so, _TAIL).wait()
                out_desc(j - 1, sprev, soprev, _CB).wait()

        @pl.when(j % 2 == 0)
        def _():
            step(x0, si0, s0, so0, x1, si1, s1, so1)

        @pl.when(j % 2 == 1)
        def _():
            step(x1, si1, s1, so1, x0, si0, s0, so0)

    return pl.pallas_call(
        body,
        grid=(_NGR,),
        in_specs=[pl.BlockSpec(memory_space=pl.ANY)],
        out_specs=pl.BlockSpec(memory_space=pl.ANY),
        out_shape=jax.ShapeDtypeStruct((_IV, 128), jnp.float32),
        scratch_shapes=[
            pltpu.VMEM((_E, _CB), jnp.float32),
            pltpu.VMEM((_E, _CB), jnp.float32),
            pltpu.VMEM((_CB, 128), jnp.float32),
            pltpu.VMEM((_CB, 128), jnp.float32),
            pltpu.SemaphoreType.DMA,
            pltpu.SemaphoreType.DMA,
            pltpu.SemaphoreType.DMA,
            pltpu.SemaphoreType.DMA,
        ],
    )(tabT)


def _fix_tail(tab128, tail64):
    def body(out_alias_hbm, x_ref, out_hbm, st, sem):
        st[:, 0:_E] = x_ref[...]
        cp = pltpu.make_async_copy(
            st, out_hbm.at[pl.ds(_IV - _REM, _REM), :], sem)
        cp.start()
        cp.wait()

    return pl.pallas_call(
        body,
        grid=(1,),
        in_specs=[
            pl.BlockSpec(memory_space=pl.ANY),
            pl.BlockSpec((_REM, _E), lambda i: (0, 0)),
        ],
        out_specs=pl.BlockSpec(memory_space=pl.ANY),
        out_shape=jax.ShapeDtypeStruct((_IV, 128), jnp.float32),
        input_output_aliases={0: 0},
        scratch_shapes=[
            pltpu.VMEM((_REM, 128), jnp.float32),
            pltpu.SemaphoreType.DMA,
        ],
    )(tab128, tail64)


def _embed_sum_sc(idx3, emb_table):
    mesh = plsc.VectorSubcoreMesh(core_axis_name="c", subcore_axis_name="s")

    @functools.partial(
        pl.kernel,
        out_type=jax.ShapeDtypeStruct((_B, _E), jnp.float32),
        mesh=mesh,
        scratch_types=[
            pltpu.VMEM((2, _NSUB, _SUBLEN), jnp.int32),
            pltpu.VMEM((2, _CHUNK * _L, _E), jnp.float32),
            pltpu.VMEM((_BPW, _E), jnp.float32),
            pltpu.SemaphoreType.DMA,
            pltpu.SemaphoreType.DMA,
        ],
        compiler_params=pltpu.CompilerParams(use_tc_tiling_on_sc=False),
    )
    def k(idx_hbm, table_hbm, out_hbm, idx_v, rows_v, out_v, sem0, sem1):
        wid = lax.axis_index("s") * 2 + lax.axis_index("c")
        row0 = wid * _T

        def fire(t, buf, sem):
            pltpu.sync_copy(idx_hbm.at[row0 + t], idx_v.at[buf])
            for j in range(_NSUB):
                pltpu.async_copy(
                    table_hbm.at[idx_v.at[buf, j]],
                    rows_v.at[buf, pl.ds(j * _SUBLEN, _SUBLEN)],
                    sem,
                )

        def drain(buf, sem):
            for j in range(_NSUB):
                pltpu.make_async_copy(
                    table_hbm.at[idx_v.at[buf, j]],
                    rows_v.at[buf, pl.ds(j * _SUBLEN, _SUBLEN)],
                    sem,
                ).wait()

        def accum(t, buf):
            for c in range(_CHUNK):
                base = c * _L

                def body(jj, accs):
                    j = base + jj * _UNROLL
                    new = []
                    for g in range(_GROUPS):
                        a = accs[g]
                        for u in range(_UNROLL):
                            a = a + rows_v[buf, j + u, pl.ds(g * 16, 16)]
                        new.append(a)
                    return tuple(new)

                zero = jnp.zeros((16,), jnp.float32)
                accs = lax.fori_loop(0, _L // _UNROLL, body, (zero,) * _GROUPS)
                for g in range(_GROUPS):
                    out_v[t * _CHUNK + c, pl.ds(g * 16, 16)] = accs[g]

        fire(0, 0, sem0)

        def pair(i, carry):
            t0 = 2 * i
            fire(t0 + 1, 1, sem1)
            drain(0, sem0)
            accum(t0, 0)

            @pl.when(t0 + 2 < _T)
            def _():
                fire(t0 + 2, 0, sem0)

            drain(1, sem1)
            accum(t0 + 1, 1)
            return carry

        lax.fori_loop(0, _T // 2, pair, 0)
        pltpu.sync_copy(out_v, out_hbm.at[pl.ds(wid * _BPW, _BPW)])

    return k(idx3, emb_table)


def _heads_tc(hid, w_outT, b_out2, w_val2, b_val2, u):
    blk = 512
    grid = _B // blk

    def body(hid_ref, w_ref, b_ref, wv_ref, bv_ref, u_ref,
             comms_ref, probs_ref, val_ref):
        h = hid_ref[...]
        logits = jnp.dot(h, w_ref[...], preferred_element_type=jnp.float32)
        logits = logits + b_ref[...]
        p = jax.nn.sigmoid(logits)
        probs_ref[...] = p
        comms_ref[...] = (u_ref[...] < p).astype(jnp.float32)
        v = jnp.sum(h * wv_ref[...], axis=1, keepdims=True) + bv_ref[0, 0]
        val_ref[...] = v

    return pl.pallas_call(
        body,
        grid=(grid,),
        in_specs=[
            pl.BlockSpec((blk, _E), lambda i: (i, 0)),
            pl.BlockSpec((_E, _OV), lambda i: (0, 0)),
            pl.BlockSpec((1, _OV), lambda i: (0, 0)),
            pl.BlockSpec((1, _E), lambda i: (0, 0)),
            pl.BlockSpec(memory_space=pltpu.SMEM),
            pl.BlockSpec((blk, _OV), lambda i: (i, 0)),
        ],
        out_specs=[
            pl.BlockSpec((blk, _OV), lambda i: (i, 0)),
            pl.BlockSpec((blk, _OV), lambda i: (i, 0)),
            pl.BlockSpec((blk, 1), lambda i: (i, 0)),
        ],
        out_shape=[
            jax.ShapeDtypeStruct((_B, _OV), jnp.float32),
            jax.ShapeDtypeStruct((_B, _OV), jnp.float32),
            jax.ShapeDtypeStruct((_B, 1), jnp.float32),
        ],
    )(hid, w_outT, b_out2, w_val2, b_val2, u)


def kernel(goldstandard, emb_table, W_out, b_out, W_val, b_val):
    idx3 = (goldstandard * 2).reshape(_B // _CHUNK, _NSUB, _SUBLEN)
    tab128 = _relayout_tc(emb_table.T)
    tab128 = _fix_tail(tab128, emb_table[_IV - _REM:, :])
    table_lin = tab128.reshape(2 * _IV, _E)
    hid = _embed_sum_sc(idx3, table_lin)
    u = jax.random.uniform(jax.random.key(1), (_B, _OV), jnp.float32)
    comms, probs, value = _heads_tc(
        hid,
        W_out.T,
        b_out.reshape(1, _OV),
        W_val,
        b_val.reshape(1, 1),
        u,
    )
    return comms, probs, value


# speedup vs baseline: 6.1257x; 1.8492x over previous
import functools

import jax
import jax.numpy as jnp
from jax import lax
from jax.experimental import pallas as pl
from jax.experimental.pallas import tpu as pltpu
from jax.experimental.pallas import tpu_sc as plsc

_B, _L, _E, _OV = 16384, 200, 64, 256
_IV = 1_000_000

_NW = 32
_BPW = _B // _NW
_CHUNK = 2
_T = _BPW // _CHUNK
_SUBLEN = 100
_NSUB = (_CHUNK * _L) // _SUBLEN
_GROUPS = _E // 16
_UNROLL = 8


_CB = 16384
_NGR = 62
_TAIL = 512
_REM = _IV - (_NGR - 1) * _CB - _TAIL


def _relayout_tc(tabT):
    def body(in_hbm, out_hbm, x0, x1, s0, s1, si0, si1, so0, so1):
        j = pl.program_id(0)

        def in_desc(jj, xbuf, sem, n):
            return pltpu.make_async_copy(
                in_hbm.at[:, pl.ds(jj * _CB, n)], xbuf.at[:, pl.ds(0, n)], sem)

        def out_desc(jj, sbuf, sem, n):
            return pltpu.make_async_copy(
                sbuf.at[pl.ds(0, n), :], out_hbm.at[pl.ds(jj * _CB, n), :], sem)

        def step(x, si, s, so, xn, sin_, sprev, soprev):
            @pl.when(j == 0)
            def _():
                in_desc(0, x, si, _CB).start()

            @pl.when(j < _NGR - 1)
            def _():
                in_desc(j, x, si, _CB).wait()

            @pl.when(j == _NGR - 1)
            def _():
                in_desc(j, x, si, _TAIL).wait()

            @pl.when(j + 1 < _NGR - 1)
            def _():
                in_desc(j + 1, xn, sin_, _CB).start()

            @pl.when(j + 1 == _NGR - 1)
            def _():
                in_desc(j + 1, xn, sin_, _TAIL).start()

            @pl.when(j >= 2)
            def _():
                out_desc(j - 2, s, so, _CB).wait()

            @pl.when(j < _NGR - 1)
            def _():
                s[:, 0:_E] = jnp.transpose(x[...], (1, 0))
                out_desc(j, s, so, _CB).start()

            @pl.when(j == _NGR - 1)
            def _():
                s[0:_TAIL, 0:_E] = jnp.transpose(x[:, 0:_TAIL], (1, 0))
                out_desc(j, s, so, _TAIL).start()
                out_desc(j, s, so, _TAIL).wait()
                out_desc(j - 1, sprev, soprev, _CB).wait()

        @pl.when(j % 2 == 0)
        def _():
            step(x0, si0, s0, so0, x1, si1, s1, so1)

        @pl.when(j % 2 == 1)
        def _():
            step(x1, si1, s1, so1, x0, si0, s0, so0)

    return pl.pallas_call(
        body,
        grid=(_NGR,),
        in_specs=[pl.BlockSpec(memory_space=pl.ANY)],
        out_specs=pl.BlockSpec(memory_space=pl.ANY),
        out_shape=jax.ShapeDtypeStruct((_IV, 128), jnp.float32),
        scratch_shapes=[
            pltpu.VMEM((_E, _CB), jnp.float32),
            pltpu.VMEM((_E, _CB), jnp.float32),
            pltpu.VMEM((_CB, 128), jnp.float32),
            pltpu.VMEM((_CB, 128), jnp.float32),
            pltpu.SemaphoreType.DMA,
            pltpu.SemaphoreType.DMA,
            pltpu.SemaphoreType.DMA,
            pltpu.SemaphoreType.DMA,
        ],
    )(tabT)


def _fix_tail(tab128, tail64):
    def body(out_alias_hbm, x_ref, out_hbm, st, sem):
        st[:, 0:_E] = x_ref[...]
        cp = pltpu.make_async_copy(
            st, out_hbm.at[pl.ds(_IV - _REM, _REM), :], sem)
        cp.start()
        cp.wait()

    return pl.pallas_call(
        body,
        grid=(1,),
        in_specs=[
            pl.BlockSpec(memory_space=pl.ANY),
            pl.BlockSpec((_REM, _E), lambda i: (0, 0)),
        ],
        out_specs=pl.BlockSpec(memory_space=pl.ANY),
        out_shape=jax.ShapeDtypeStruct((_IV, 128), jnp.float32),
        input_output_aliases={0: 0},
        scratch_shapes=[
            pltpu.VMEM((_REM, 128), jnp.float32),
            pltpu.SemaphoreType.DMA,
        ],
    )(tab128, tail64)


def _embed_sum_sc(idx3, emb_table):
    mesh = plsc.VectorSubcoreMesh(core_axis_name="c", subcore_axis_name="s")

    @functools.partial(
        pl.kernel,
        out_type=jax.ShapeDtypeStruct((_B, _E), jnp.float32),
        mesh=mesh,
        scratch_types=[
            pltpu.VMEM((2, _NSUB, _SUBLEN), jnp.int32),
            pltpu.VMEM((2, _CHUNK * _L, _E), jnp.float32),
            pltpu.VMEM((_BPW, _E), jnp.float32),
            pltpu.SemaphoreType.DMA,
            pltpu.SemaphoreType.DMA,
        ],
        compiler_params=pltpu.CompilerParams(use_tc_tiling_on_sc=False),
    )
    def k(idx_hbm, table_hbm, out_hbm, idx_v, rows_v, out_v, sem0, sem1):
        wid = lax.axis_index("s") * 2 + lax.axis_index("c")
        row0 = wid * _T

        def fire(t, buf, sem):
            pltpu.sync_copy(idx_hbm.at[row0 + t], idx_v.at[buf])
            for j in range(_NSUB):
                pltpu.async_copy(
                    table_hbm.at[idx_v.at[buf, j]],
                    rows_v.at[buf, pl.ds(j * _SUBLEN, _SUBLEN)],
                    sem,
                )

        def drain(buf, sem):
            for j in range(_NSUB):
                pltpu.make_async_copy(
                    table_hbm.at[idx_v.at[buf, j]],
                    rows_v.at[buf, pl.ds(j * _SUBLEN, _SUBLEN)],
                    sem,
                ).wait()

        def accum(t, buf):
            for c in range(_CHUNK):
                base = c * _L

                def body(jj, accs):
                    j = base + jj * _UNROLL
                    new = []
                    for g in range(_GROUPS):
                        a = accs[g]
                        for u in range(_UNROLL):
                            a = a + rows_v[buf, j + u, pl.ds(g * 16, 16)]
                        new.append(a)
                    return tuple(new)

                zero = jnp.zeros((16,), jnp.float32)
                accs = lax.fori_loop(0, _L // _UNROLL, body, (zero,) * _GROUPS)
                for g in range(_GROUPS):
                    out_v[t * _CHUNK + c, pl.ds(g * 16, 16)] = accs[g]

        fire(0, 0, sem0)

        def pair(i, carry):
            t0 = 2 * i
            fire(t0 + 1, 1, sem1)
            drain(0, sem0)
            accum(t0, 0)

            @pl.when(t0 + 2 < _T)
            def _():
                fire(t0 + 2, 0, sem0)

            drain(1, sem1)
            accum(t0 + 1, 1)
            return carry

        lax.fori_loop(0, _T // 2, pair, 0)
        pltpu.sync_copy(out_v, out_hbm.at[pl.ds(wid * _BPW, _BPW)])

    return k(idx3, emb_table)


def _heads_tc(hid, w_outT, b_out2, w_val2, b_val2, u):
    blk = 512
    grid = _B // blk

    def body(hid_ref, w_ref, b_ref, wv_ref, bv_ref, u_ref,
             comms_ref, probs_ref, val_ref):
        h = hid_ref[...]
        logits = jnp.dot(h, w_ref[...], preferred_element_type=jnp.float32)
        logits = logits + b_ref[...]
        p = jax.nn.sigmoid(logits)
        probs_ref[...] = p
        comms_ref[...] = (u_ref[...] < p).astype(jnp.float32)
        v = jnp.sum(h * wv_ref[...], axis=1, keepdims=True) + bv_ref[0, 0]
        val_ref[...] = v

    return pl.pallas_call(
        body,
        grid=(grid,),
        in_specs=[
            pl.BlockSpec((blk, _E), lambda i: (i, 0)),
            pl.BlockSpec((_E, _OV), lambda i: (0, 0)),
            pl.BlockSpec((1, _OV), lambda i: (0, 0)),
            pl.BlockSpec((1, _E), lambda i: (0, 0)),
            pl.BlockSpec(memory_space=pltpu.SMEM),
            pl.BlockSpec((blk, _OV), lambda i: (i, 0)),
        ],
        out_specs=[
            pl.BlockSpec((blk, _OV), lambda i: (i, 0)),
            pl.BlockSpec((blk, _OV), lambda i: (i, 0)),
            pl.BlockSpec((blk, 1), lambda i: (i, 0)),
        ],
        out_shape=[
            jax.ShapeDtypeStruct((_B, _OV), jnp.float32),
            jax.ShapeDtypeStruct((_B, _OV), jnp.float32),
            jax.ShapeDtypeStruct((_B, 1), jnp.float32),
        ],
    )(hid, w_outT, b_out2, w_val2, b_val2, u)


def kernel(goldstandard, emb_table, W_out, b_out, W_val, b_val):
    idx3 = (goldstandard * 2).reshape(_B // _CHUNK, _NSUB, _SUBLEN)
    tab128 = _relayout_tc(emb_table.T)
    tab128 = _fix_tail(tab128, emb_table[_IV - _REM:, :])
    table_lin = tab128.reshape(2 * _IV, _E)
    hid = _embed_sum_sc(idx3, table_lin)
    u = jax.random.uniform(jax.random.key(1), (_B, _OV), jnp.float32)
    comms, probs, value = _heads_tc(
        hid,
        W_out.T,
        b_out.reshape(1, _OV),
        W_val,
        b_val.reshape(1, 1),
        u,
    )
    return comms, probs, value


# speedup vs baseline: 6.5773x; 1.0737x over previous
import functools

import jax
import jax.numpy as jnp
from jax import lax
from jax.experimental import pallas as pl
from jax.experimental.pallas import tpu as pltpu
from jax.experimental.pallas import tpu_sc as plsc

_B, _L, _E, _OV = 16384, 200, 64, 256
_IV = 1_000_000

_NW = 32
_BPW = _B // _NW
_CHUNK = 2
_T = _BPW // _CHUNK
_SUBLEN = 80
_NSUB = (_CHUNK * _L) // _SUBLEN
_GROUPS = _E // 16
_UNROLL = 8


_CB = 16384
_NGR = 62
_TAIL = 512
_REM = _IV - (_NGR - 1) * _CB - _TAIL


def _relayout_tc(tabT):
    def body(in_hbm, out_hbm, x0, x1, s0, s1, si0, si1, so0, so1):
        j = pl.program_id(0)

        def in_desc(jj, xbuf, sem, n):
            return pltpu.make_async_copy(
                in_hbm.at[:, pl.ds(jj * _CB, n)], xbuf.at[:, pl.ds(0, n)], sem)

        def out_desc(jj, sbuf, sem, n):
            return pltpu.make_async_copy(
                sbuf.at[pl.ds(0, n), :], out_hbm.at[pl.ds(jj * _CB, n), :], sem)

        def step(x, si, s, so, xn, sin_, sprev, soprev):
            @pl.when(j == 0)
            def _():
                in_desc(0, x, si, _CB).start()

            @pl.when(j < _NGR - 1)
            def _():
                in_desc(j, x, si, _CB).wait()

            @pl.when(j == _NGR - 1)
            def _():
                in_desc(j, x, si, _TAIL).wait()

            @pl.when(j + 1 < _NGR - 1)
            def _():
                in_desc(j + 1, xn, sin_, _CB).start()

            @pl.when(j + 1 == _NGR - 1)
            def _():
                in_desc(j + 1, xn, sin_, _TAIL).start()

            @pl.when(j >= 2)
            def _():
                out_desc(j - 2, s, so, _CB).wait()

            @pl.when(j < _NGR - 1)
            def _():
                s[:, 0:_E] = jnp.transpose(x[...], (1, 0))
                out_desc(j, s, so, _CB).start()

            @pl.when(j == _NGR - 1)
            def _():
                s[0:_TAIL, 0:_E] = jnp.transpose(x[:, 0:_TAIL], (1, 0))
                out_desc(j, s, so, _TAIL).start()
                out_desc(j, s, so, _TAIL).wait()
                out_desc(j - 1, sprev, soprev, _CB).wait()

        @pl.when(j % 2 == 0)
        def _():
            step(x0, si0, s0, so0, x1, si1, s1, so1)

        @pl.when(j % 2 == 1)
        def _():
            step(x1, si1, s1, so1, x0, si0, s0, so0)

    return pl.pallas_call(
        body,
        grid=(_NGR,),
        in_specs=[pl.BlockSpec(memory_space=pl.ANY)],
        out_specs=pl.BlockSpec(memory_space=pl.ANY),
        out_shape=jax.ShapeDtypeStruct((_IV, 128), jnp.float32),
        scratch_shapes=[
            pltpu.VMEM((_E, _CB), jnp.float32),
            pltpu.VMEM((_E, _CB), jnp.float32),
            pltpu.VMEM((_CB, 128), jnp.float32),
            pltpu.VMEM((_CB, 128), jnp.float32),
            pltpu.SemaphoreType.DMA,
            pltpu.SemaphoreType.DMA,
            pltpu.SemaphoreType.DMA,
            pltpu.SemaphoreType.DMA,
        ],
    )(tabT)


def _fix_tail(tab128, tail64):
    def body(out_alias_hbm, x_ref, out_hbm, st, sem):
        st[:, 0:_E] = x_ref[...]
        cp = pltpu.make_async_copy(
            st, out_hbm.at[pl.ds(_IV - _REM, _REM), :], sem)
        cp.start()
        cp.wait()

    return pl.pallas_call(
        body,
        grid=(1,),
        in_specs=[
            pl.BlockSpec(memory_space=pl.ANY),
            pl.BlockSpec((_REM, _E), lambda i: (0, 0)),
        ],
        out_specs=pl.BlockSpec(memory_space=pl.ANY),
        out_shape=jax.ShapeDtypeStruct((_IV, 128), jnp.float32),
        input_output_aliases={0: 0},
        scratch_shapes=[
            pltpu.VMEM((_REM, 128), jnp.float32),
            pltpu.SemaphoreType.DMA,
        ],
    )(tab128, tail64)


def _embed_sum_sc(idx3, emb_table):
    mesh = plsc.VectorSubcoreMesh(core_axis_name="c", subcore_axis_name="s")

    @functools.partial(
        pl.kernel,
        out_type=jax.ShapeDtypeStruct((_B, _E), jnp.float32),
        mesh=mesh,
        scratch_types=[
            pltpu.VMEM((2, _NSUB, _SUBLEN), jnp.int32),
            pltpu.VMEM((2, _CHUNK * _L, _E), jnp.float32),
            pltpu.VMEM((_BPW, _E), jnp.float32),
            pltpu.SemaphoreType.DMA,
            pltpu.SemaphoreType.DMA,
        ],
        compiler_params=pltpu.CompilerParams(use_tc_tiling_on_sc=False),
    )
    def k(idx_hbm, table_hbm, out_hbm, idx_v, rows_v, out_v, sem0, sem1):
        wid = lax.axis_index("s") * 2 + lax.axis_index("c")
        row0 = wid * _T

        def fire(t, buf, sem):
            pltpu.sync_copy(idx_hbm.at[row0 + t], idx_v.at[buf])
            for j in range(_NSUB):
                pltpu.async_copy(
                    table_hbm.at[idx_v.at[buf, j]],
                    rows_v.at[buf, pl.ds(j * _SUBLEN, _SUBLEN)],
                    sem,
                )

        def drain(buf, sem):
            for j in range(_NSUB):
                pltpu.make_async_copy(
                    table_hbm.at[idx_v.at[buf, j]],
                    rows_v.at[buf, pl.ds(j * _SUBLEN, _SUBLEN)],
                    sem,
                ).wait()

        def accum(t, buf):
            for c in range(_CHUNK):
                base = c * _L

                def body(jj, accs):
                    j = base + jj * _UNROLL
                    new = []
                    for g in range(_GROUPS):
                        a = accs[g]
                        for u in range(_UNROLL):
                            a = a + rows_v[buf, j + u, pl.ds(g * 16, 16)]
                        new.append(a)
                    return tuple(new)

                zero = jnp.zeros((16,), jnp.float32)
                accs = lax.fori_loop(0, _L // _UNROLL, body, (zero,) * _GROUPS)
                for g in range(_GROUPS):
                    out_v[t * _CHUNK + c, pl.ds(g * 16, 16)] = accs[g]

        fire(0, 0, sem0)

        def pair(i, carry):
            t0 = 2 * i
            fire(t0 + 1, 1, sem1)
            drain(0, sem0)
            accum(t0, 0)

            @pl.when(t0 + 2 < _T)
            def _():
                fire(t0 + 2, 0, sem0)

            drain(1, sem1)
            accum(t0 + 1, 1)
            return carry

        lax.fori_loop(0, _T // 2, pair, 0)
        pltpu.sync_copy(out_v, out_hbm.at[pl.ds(wid * _BPW, _BPW)])

    return k(idx3, emb_table)


def _heads_tc(hid, w_outT, b_out2, w_val2, b_val2, u):
    blk = 2048
    grid = _B // blk

    def body(hid_ref, w_ref, b_ref, wv_ref, bv_ref, u_ref,
             comms_ref, probs_ref, val_ref):
        h = hid_ref[...]
        logits = jnp.dot(h, w_ref[...], preferred_element_type=jnp.float32)
        logits = logits + b_ref[...]
        p = jax.nn.sigmoid(logits)
        probs_ref[...] = p
        comms_ref[...] = (u_ref[...] < p).astype(jnp.float32)
        v = jnp.sum(h * wv_ref[...], axis=1, keepdims=True) + bv_ref[0, 0]
        val_ref[...] = v

    return pl.pallas_call(
        body,
        grid=(grid,),
        in_specs=[
            pl.BlockSpec((blk, _E), lambda i: (i, 0)),
            pl.BlockSpec((_E, _OV), lambda i: (0, 0)),
            pl.BlockSpec((1, _OV), lambda i: (0, 0)),
            pl.BlockSpec((1, _E), lambda i: (0, 0)),
            pl.BlockSpec(memory_space=pltpu.SMEM),
            pl.BlockSpec((blk, _OV), lambda i: (i, 0)),
        ],
        out_specs=[
            pl.BlockSpec((blk, _OV), lambda i: (i, 0)),
            pl.BlockSpec((blk, _OV), lambda i: (i, 0)),
            pl.BlockSpec((blk, 1), lambda i: (i, 0)),
        ],
        out_shape=[
            jax.ShapeDtypeStruct((_B, _OV), jnp.float32),
            jax.ShapeDtypeStruct((_B, _OV), jnp.float32),
            jax.ShapeDtypeStruct((_B, 1), jnp.float32),
        ],
    )(hid, w_outT, b_out2, w_val2, b_val2, u)


def kernel(goldstandard, emb_table, W_out, b_out, W_val, b_val):
    idx3 = (goldstandard * 2).reshape(_B // _CHUNK, _NSUB, _SUBLEN)
    tab128 = _relayout_tc(emb_table.T)
    tab128 = _fix_tail(tab128, emb_table[_IV - _REM:, :])
    table_lin = tab128.reshape(2 * _IV, _E)
    hid = _embed_sum_sc(idx3, table_lin)
    u = jax.random.uniform(jax.random.key(1), (_B, _OV), jnp.float32)
    comms, probs, value = _heads_tc(
        hid,
        W_out.T,
        b_out.reshape(1, _OV),
        W_val,
        b_val.reshape(1, 1),
        u,
    )
    return comms, probs, value
